# Initial kernel scaffold; baseline (speedup 1.0000x reference)
import jax
import jax.numpy as jnp
from jax.experimental import pallas as pl


def kernel(x, edge_index, W_l, b_l, W_r):
    raise NotImplementedError("write your pallas kernel here")


# speedup vs baseline: 7.5703x; 7.5703x over previous
import functools

import jax
import jax.numpy as jnp
from jax import lax
from jax.experimental import pallas as pl
from jax.experimental.pallas import tpu as pltpu
from jax.experimental.pallas import tpu_sc as plsc

N = 10000
E = 320000
C = 128
C_AUG = 144
NC, NS = 2, 16
NW = NC * NS
E_TILE = E // NW
CHUNK = 125
NCHUNK = E_TILE // CHUNK
N_PAD = 10240
ROWS_TILE = N_PAD // NS

_sc_mesh = plsc.VectorSubcoreMesh(core_axis_name="c", subcore_axis_name="s")


@functools.partial(
    pl.kernel,
    mesh=_sc_mesh,
    out_type=jax.ShapeDtypeStruct((NC, N_PAD, C_AUG), jnp.float32),
    scratch_types=[
        pltpu.VMEM((NCHUNK, CHUNK), jnp.int32),
        pltpu.VMEM((NCHUNK, CHUNK), jnp.int32),
        pltpu.VMEM((CHUNK, C_AUG), jnp.float32),
        pltpu.VMEM_SHARED((N_PAD, C_AUG), jnp.float32),
        pltpu.SemaphoreType.DMA,
    ],
    compiler_params=pltpu.CompilerParams(use_tc_tiling_on_sc=False),
)
def _sc_aggregate(xaug_hbm, src_hbm, dst_hbm, zeros_hbm, out_hbm,
                  src_v, dst_v, buf, acc_sh, sem):
    c = lax.axis_index("c")
    s = lax.axis_index("s")
    pltpu.sync_copy(zeros_hbm.at[pl.ds(s * ROWS_TILE, ROWS_TILE)],
                    acc_sh.at[pl.ds(s * ROWS_TILE, ROWS_TILE)])
    pltpu.sync_copy(src_hbm.at[c, s], src_v)
    pltpu.sync_copy(dst_hbm.at[c, s], dst_v)
    plsc.subcore_barrier()

    def body(j, _):
        pltpu.async_copy(xaug_hbm.at[src_v.at[j]], buf, sem).wait()
        pltpu.sync_copy(buf, acc_sh.at[dst_v.at[j]], add=True)
        return ()

    lax.fori_loop(0, NCHUNK, body, ())
    plsc.subcore_barrier()
    pltpu.sync_copy(acc_sh.at[pl.ds(s * ROWS_TILE, ROWS_TILE)],
                    out_hbm.at[c].at[pl.ds(s * ROWS_TILE, ROWS_TILE)])


BLK = 1000


def _tc_combine_body(acc_ref, x_ref, wl_ref, wr_ref, b_ref, out_ref):
    a = acc_ref[0] + acc_ref[1]
    deg = a[:, C:C + 1]
    scale = 1.0 / jnp.maximum(deg, 1.0)
    agg = a[:, :C] * scale
    dn = (((1,), (1,)), ((), ()))
    out_ref[...] = (
        lax.dot_general(agg, wl_ref[...], dn, preferred_element_type=jnp.float32)
        + lax.dot_general(x_ref[...], wr_ref[...], dn, preferred_element_type=jnp.float32)
        + b_ref[...]
    )


def _tc_combine(acc, x, W_l, W_r, b_l):
    return pl.pallas_call(
        _tc_combine_body,
        grid=(N // BLK,),
        in_specs=[
            pl.BlockSpec((NC, BLK, C_AUG), lambda i: (0, i, 0)),
            pl.BlockSpec((BLK, C), lambda i: (i, 0)),
            pl.BlockSpec((C, C), lambda i: (0, 0)),
            pl.BlockSpec((C, C), lambda i: (0, 0)),
            pl.BlockSpec((1, C), lambda i: (0, 0)),
        ],
        out_specs=pl.BlockSpec((BLK, C), lambda i: (i, 0)),
        out_shape=jax.ShapeDtypeStruct((N, C), jnp.float32),
    )(acc, x, W_l, W_r, b_l)


def kernel(x, edge_index, W_l, b_l, W_r):
    x_aug = jnp.concatenate(
        [x, jnp.ones((N, 1), jnp.float32), jnp.zeros((N, C_AUG - C - 1), jnp.float32)],
        axis=1)
    src = edge_index[0].reshape(NC, NS, NCHUNK, CHUNK)
    dst = edge_index[1].reshape(NC, NS, NCHUNK, CHUNK)
    zeros = jnp.zeros((N_PAD, C_AUG), jnp.float32)
    acc = _sc_aggregate(x_aug, src, dst, zeros)
    return _tc_combine(acc[:, :N, :], x, W_l, W_r, b_l.reshape(1, C))


# speedup vs baseline: 8.8012x; 1.1626x over previous
import functools

import jax
import jax.numpy as jnp
from jax import lax
from jax.experimental import pallas as pl
from jax.experimental.pallas import tpu as pltpu
from jax.experimental.pallas import tpu_sc as plsc

N = 10000
E = 320000
C = 128
C_AUG = 144
NC, NS = 2, 16
NW = NC * NS
E_TILE = E // NW
CHUNK = 50
NCHUNK = E_TILE // CHUNK
N_PAD = 10240
ROWS_TILE = N_PAD // NS

_sc_mesh = plsc.VectorSubcoreMesh(core_axis_name="c", subcore_axis_name="s")


@functools.partial(
    pl.kernel,
    mesh=_sc_mesh,
    out_type=jax.ShapeDtypeStruct((NC, N_PAD, C_AUG), jnp.float32),
    scratch_types=[
        pltpu.VMEM((NCHUNK, CHUNK), jnp.int32),
        pltpu.VMEM((NCHUNK, CHUNK), jnp.int32),
        pltpu.VMEM((CHUNK, C_AUG), jnp.float32),
        pltpu.VMEM((CHUNK, C_AUG), jnp.float32),
        pltpu.VMEM_SHARED((N_PAD, C_AUG), jnp.float32),
        pltpu.SemaphoreType.DMA,
        pltpu.SemaphoreType.DMA,
    ],
    compiler_params=pltpu.CompilerParams(use_tc_tiling_on_sc=False),
)
def _sc_aggregate(xaug_hbm, src_hbm, dst_hbm, zeros_hbm, out_hbm,
                  src_v, dst_v, buf0, buf1, acc_sh, sem0, sem1):
    c = lax.axis_index("c")
    s = lax.axis_index("s")
    pltpu.sync_copy(zeros_hbm.at[pl.ds(s * ROWS_TILE, ROWS_TILE)],
                    acc_sh.at[pl.ds(s * ROWS_TILE, ROWS_TILE)])
    pltpu.sync_copy(src_hbm.at[c, s], src_v)
    pltpu.sync_copy(dst_hbm.at[c, s], dst_v)
    plsc.subcore_barrier()

    def gather(j, buf, sem):
        pltpu.async_copy(xaug_hbm.at[src_v.at[j]], buf, sem)

    def gwait(buf, sem):
        pltpu.make_async_copy(xaug_hbm.at[src_v.at[0]], buf, sem).wait()

    gather(0, buf0, sem0)

    def body(i, _):
        j0 = 2 * i
        gather(lax.rem(j0 + 1, NCHUNK), buf1, sem1)
        gwait(buf0, sem0)
        pltpu.sync_copy(buf0, acc_sh.at[dst_v.at[j0]], add=True)
        gather(lax.rem(j0 + 2, NCHUNK), buf0, sem0)
        gwait(buf1, sem1)
        pltpu.sync_copy(buf1, acc_sh.at[dst_v.at[j0 + 1]], add=True)
        return ()

    lax.fori_loop(0, NCHUNK // 2, body, ())
    gwait(buf0, sem0)
    plsc.subcore_barrier()
    pltpu.sync_copy(acc_sh.at[pl.ds(s * ROWS_TILE, ROWS_TILE)],
                    out_hbm.at[c].at[pl.ds(s * ROWS_TILE, ROWS_TILE)])


BLK = 1000


def _tc_combine_body(acc_ref, x_ref, wl_ref, wr_ref, b_ref, out_ref):
    a = acc_ref[0] + acc_ref[1]
    deg = a[:, C:C + 1]
    scale = 1.0 / jnp.maximum(deg, 1.0)
    agg = a[:, :C] * scale
    dn = (((1,), (1,)), ((), ()))
    out_ref[...] = (
        lax.dot_general(agg, wl_ref[...], dn, preferred_element_type=jnp.float32)
        + lax.dot_general(x_ref[...], wr_ref[...], dn, preferred_element_type=jnp.float32)
        + b_ref[...]
    )


def _tc_combine(acc, x, W_l, W_r, b_l):
    return pl.pallas_call(
        _tc_combine_body,
        grid=(N // BLK,),
        in_specs=[
            pl.BlockSpec((NC, BLK, C_AUG), lambda i: (0, i, 0)),
            pl.BlockSpec((BLK, C), lambda i: (i, 0)),
            pl.BlockSpec((C, C), lambda i: (0, 0)),
            pl.BlockSpec((C, C), lambda i: (0, 0)),
            pl.BlockSpec((1, C), lambda i: (0, 0)),
        ],
        out_specs=pl.BlockSpec((BLK, C), lambda i: (i, 0)),
        out_shape=jax.ShapeDtypeStruct((N, C), jnp.float32),
    )(acc, x, W_l, W_r, b_l)


def kernel(x, edge_index, W_l, b_l, W_r):
    x_aug = jnp.concatenate(
        [x, jnp.ones((N, 1), jnp.float32), jnp.zeros((N, C_AUG - C - 1), jnp.float32)],
        axis=1)
    src = edge_index[0].reshape(NC, NS, NCHUNK, CHUNK)
    dst = edge_index[1].reshape(NC, NS, NCHUNK, CHUNK)
    zeros = jnp.zeros((N_PAD, C_AUG), jnp.float32)
    acc = _sc_aggregate(x_aug, src, dst, zeros)
    return _tc_combine(acc, x, W_l, W_r, b_l.reshape(1, C))
